# baseline (speedup 1.0000x reference)
import functools

import jax
import jax.numpy as jnp
from jax import lax
from jax.experimental import pallas as pl
from jax.experimental.pallas import tpu as pltpu
from jax.experimental.pallas import tpu_sc as plsc

N = 10000
E = 160000
F = 20000
D = 3
T = 16
S = 16
R = 1.1
B = 8

DELTA = 2.0 * R / (S - 1)
NW = 32
N_PAD = 10240
CHUNK = 512
E_CHUNKS = 10
F_CHUNKS = 2
E_PAD = NW * E_CHUNKS * CHUNK
F_PAD = NW * F_CHUNKS * CHUNK
N_CHUNK = 320


def _pack_body(x_ref, v_ref, w_ref, b_ref, out_ref):
    xv = x_ref[...]
    vv = v_ref[...]
    nh = (xv[:, 0:1] * vv[0:1, :]
          + xv[:, 1:2] * vv[1:2, :]
          + xv[:, 2:3] * vv[2:3, :])
    out_ref[:, 0:16] = nh
    out_ref[:, 16:17] = w_ref[...]
    out_ref[:, 17:18] = b_ref[...]
    out_ref[:, 18:32] = jnp.zeros((N_PAD, 14), jnp.float32)


def _fin_body(hist_ref, out_ref):
    s = jnp.sum(hist_ref[...], axis=0)
    h2 = s[0:128, :]
    g2 = s[128:256, :]
    r = lax.broadcasted_iota(jnp.int32, (128, 128), 0)
    c = lax.broadcasted_iota(jnp.int32, (128, 128), 1)
    m = ((r >> 4) == (c >> 4)) & ((c & 15) < (r & 15))
    out_ref[...] = h2 + jnp.dot(m.astype(jnp.float32), g2,
                                preferred_element_type=jnp.float32)


def _sc_body(packed_hbm, ei0_hbm, ei1_hbm, fa0_hbm, fa1_hbm, fa2_hbm,
             out_hbm, hist_v, stage_v, rows_v, node_v, idx_v, sems, sem_n):
    cid = lax.axis_index("c")
    sid = lax.axis_index("s")
    wid = sid * 2 + cid
    lane = lax.iota(jnp.int32, 16)

    nbase = pl.multiple_of(wid * N_CHUNK, N_CHUNK)
    ncp = pltpu.async_copy(packed_hbm.at[pl.ds(nbase, N_CHUNK)],
                           node_v, sem_n)

    zero16 = jnp.zeros((16,), jnp.float32)

    def _zero(i, carry):
        hist_v[pl.ds(i * 16, 16)] = zero16
        return carry

    lax.fori_loop(0, 256, _zero, 0)

    inv = 1.0 / DELTA
    c0 = R / DELTA + 0.5
    scale = -500.0 * DELTA

    def _accum(h, sw, bv):
        u = h * inv + c0
        jf = jnp.minimum(jnp.maximum(u, 0.0), 15.0)
        j = jf.astype(jnp.int32)
        jq = j.astype(jnp.float32)
        z = jq * scale + (h * 500.0 + 500.0 * R)
        z = jnp.minimum(z, 30.0)
        wsig = sw / (1.0 + jnp.exp(z))
        idx = bv.astype(jnp.int32) + j * 16 + lane
        plsc.addupdate_scatter(hist_v, [idx], wsig)
        plsc.addupdate_scatter(hist_v, [idx + 2048], sw)

    tasks = ([("e", ch) for ch in range(E_CHUNKS)]
             + [("f", ch) for ch in range(F_CHUNKS)])

    def _fire(t, p):
        kind, ch = tasks[t]
        cps = []
        if kind == "e":
            blk = wid * E_CHUNKS + ch
            pltpu.sync_copy(ei0_hbm.at[blk], idx_v.at[p, 0])
            pltpu.sync_copy(ei1_hbm.at[blk], idx_v.at[p, 1])
            nrows = 2
        else:
            blk = wid * F_CHUNKS + ch
            pltpu.sync_copy(fa0_hbm.at[blk], idx_v.at[p, 0])
            pltpu.sync_copy(fa1_hbm.at[blk], idx_v.at[p, 1])
            pltpu.sync_copy(fa2_hbm.at[blk], idx_v.at[p, 2])
            nrows = 3
        for r in range(nrows):
            for a in range(4):
                cps.append(pltpu.async_copy(
                    packed_hbm.at[idx_v.at[p, r, a]],
                    rows_v.at[p, r, pl.ds(a * 128, 128)], sems.at[p]))
        return cps

    def _compute(t, p):
        kind, _ = tasks[t]
        if kind == "e":
            @plsc.parallel_loop(0, CHUNK, 1, unroll=8)
            def _ebody(e):
                h = jnp.minimum(rows_v[p, 0, e, 0:16], rows_v[p, 1, e, 0:16])
                s0 = rows_v[p, 0, e, 16:32]
                wm = jnp.maximum(s0, rows_v[p, 1, e, 16:32])
                zi = jnp.zeros((16,), jnp.int32)
                wv = wm.at[zi].get(mode="promise_in_bounds")
                bv = s0.at[zi + 1].get(mode="promise_in_bounds")
                _accum(h, -wv, bv)
        else:
            @plsc.parallel_loop(0, CHUNK, 1, unroll=8)
            def _fbody(e):
                h = jnp.minimum(
                    jnp.minimum(rows_v[p, 0, e, 0:16], rows_v[p, 1, e, 0:16]),
                    rows_v[p, 2, e, 0:16])
                s0 = rows_v[p, 0, e, 16:32]
                wm = jnp.maximum(jnp.maximum(s0, rows_v[p, 1, e, 16:32]),
                                 rows_v[p, 2, e, 16:32])
                zi = jnp.zeros((16,), jnp.int32)
                wv = wm.at[zi].get(mode="promise_in_bounds")
                bv = s0.at[zi + 1].get(mode="promise_in_bounds")
                _accum(h, wv, bv)

    pending = _fire(0, 0)
    for t in range(len(tasks)):
        nxt = _fire(t + 1, (t + 1) % 2) if t + 1 < len(tasks) else None
        for cp in pending:
            cp.wait()
        _compute(t, t % 2)
        pending = nxt

    ncp.wait()

    @plsc.parallel_loop(0, N_CHUNK, 1, unroll=8)
    def _nbody(e):
        h = node_v[e, 0:16]
        s0 = node_v[e, 16:32]
        zi = jnp.zeros((16,), jnp.int32)
        wv = s0.at[zi].get(mode="promise_in_bounds")
        bv = s0.at[zi + 1].get(mode="promise_in_bounds")
        _accum(h, wv, bv)

    def _stage(i, carry):
        stage_v[i, :] = hist_v[pl.ds(i * 16, 16)]
        return carry

    lax.fori_loop(0, 256, _stage, 0)
    pltpu.sync_copy(stage_v, out_hbm.at[wid])


_sc_call = pl.kernel(
    _sc_body,
    out_type=jax.ShapeDtypeStruct((NW, 256, 16), jnp.float32),
    mesh=plsc.VectorSubcoreMesh(core_axis_name="c", subcore_axis_name="s"),
    compiler_params=pltpu.CompilerParams(needs_layout_passes=False,
                                         use_tc_tiling_on_sc=False),
    scratch_types=[
        pltpu.VMEM((4096,), jnp.float32),
        pltpu.VMEM((256, 16), jnp.float32),
        pltpu.VMEM((2, 3, CHUNK, 32), jnp.float32),
        pltpu.VMEM((N_CHUNK, 32), jnp.float32),
        pltpu.VMEM((2, 3, 4, 128), jnp.int32),
        pltpu.SemaphoreType.DMA((2,)),
        pltpu.SemaphoreType.DMA,
    ],
)


@jax.jit
def kernel(x, edge_index, face, node_weights, batch, v):
    ei = edge_index.astype(jnp.int32)
    fa = face.astype(jnp.int32)

    xp = jnp.concatenate([x, jnp.zeros((N_PAD - N, D), jnp.float32)], axis=0)
    wp = jnp.concatenate([node_weights,
                          jnp.zeros((N_PAD - N,), jnp.float32)])[:, None]
    bp = jnp.concatenate([batch.astype(jnp.float32) * 256.0,
                          jnp.zeros((N_PAD - N,), jnp.float32)])[:, None]

    packed = pl.pallas_call(
        _pack_body,
        out_shape=jax.ShapeDtypeStruct((N_PAD, 32), jnp.float32),
    )(xp, v, wp, bp)

    epad = jnp.full((E_PAD - E,), N, jnp.int32)
    fpad = jnp.full((F_PAD - F,), N, jnp.int32)
    ei0 = jnp.concatenate([ei[0], epad]).reshape(NW * E_CHUNKS, 4, 128)
    ei1 = jnp.concatenate([ei[1], epad]).reshape(NW * E_CHUNKS, 4, 128)
    fa0 = jnp.concatenate([fa[0], fpad]).reshape(NW * F_CHUNKS, 4, 128)
    fa1 = jnp.concatenate([fa[1], fpad]).reshape(NW * F_CHUNKS, 4, 128)
    fa2 = jnp.concatenate([fa[2], fpad]).reshape(NW * F_CHUNKS, 4, 128)

    hist = _sc_call(packed, ei0, ei1, fa0, fa1, fa2)

    out2 = pl.pallas_call(
        _fin_body,
        out_shape=jax.ShapeDtypeStruct((128, 16), jnp.float32),
    )(hist)
    return out2.reshape(B, S, T)


# speedup vs baseline: 1.0038x; 1.0038x over previous
import functools

import jax
import jax.numpy as jnp
from jax import lax
from jax.experimental import pallas as pl
from jax.experimental.pallas import tpu as pltpu
from jax.experimental.pallas import tpu_sc as plsc

N = 10000
E = 160000
F = 20000
D = 3
T = 16
S = 16
R = 1.1
B = 8

DELTA = 2.0 * R / (S - 1)
NW = 32
N_PAD = 10240
CHUNK = 512
E_CHUNKS = 10
F_CHUNKS = 2
E_PAD = NW * E_CHUNKS * CHUNK
F_PAD = NW * F_CHUNKS * CHUNK
N_CHUNK = 320


def _pack_body(x_ref, v_ref, w_ref, b_ref, out_ref):
    xv = x_ref[...]
    vv = v_ref[...]
    nh = (xv[:, 0:1] * vv[0:1, :]
          + xv[:, 1:2] * vv[1:2, :]
          + xv[:, 2:3] * vv[2:3, :])
    out_ref[:, 0:16] = nh
    out_ref[:, 16:17] = w_ref[...]
    out_ref[:, 17:18] = b_ref[...]
    out_ref[:, 18:32] = jnp.zeros((N_PAD, 14), jnp.float32)


def _fin_body(hist_ref, out_ref):
    s = jnp.sum(hist_ref[...], axis=0)
    h2 = s[0:128, :]
    g2 = s[128:256, :]
    r = lax.broadcasted_iota(jnp.int32, (128, 128), 0)
    c = lax.broadcasted_iota(jnp.int32, (128, 128), 1)
    m = ((r >> 4) == (c >> 4)) & ((c & 15) < (r & 15))
    out_ref[...] = h2 + jnp.dot(m.astype(jnp.float32), g2,
                                preferred_element_type=jnp.float32)


def _sc_body(packed_hbm, ei0_hbm, ei1_hbm, fa0_hbm, fa1_hbm, fa2_hbm,
             out_hbm, hist_v, stage_v, rows_v, node_v, idx_v, sems, sem_n):
    cid = lax.axis_index("c")
    sid = lax.axis_index("s")
    wid = sid * 2 + cid
    lane = lax.iota(jnp.int32, 16)

    nbase = pl.multiple_of(wid * N_CHUNK, N_CHUNK)
    ncp = pltpu.async_copy(packed_hbm.at[pl.ds(nbase, N_CHUNK)],
                           node_v, sem_n)

    zero16 = jnp.zeros((16,), jnp.float32)

    def _zero(i, carry):
        hist_v[pl.ds(i * 16, 16)] = zero16
        return carry

    lax.fori_loop(0, 256, _zero, 0)

    inv = 1.0 / DELTA
    c0 = R / DELTA + 0.5
    scale = -500.0 * DELTA

    def _accum(h, sw, bv):
        u = h * inv + c0
        jf = jnp.minimum(jnp.maximum(u, 0.0), 15.0)
        j = jf.astype(jnp.int32)
        jq = j.astype(jnp.float32)
        z = jq * scale + (h * 500.0 + 500.0 * R)
        z = jnp.minimum(z, 30.0)
        wsig = sw / (1.0 + jnp.exp(z))
        idx = bv.astype(jnp.int32) + j * 16 + lane
        plsc.addupdate_scatter(hist_v, [idx], wsig)
        plsc.addupdate_scatter(hist_v, [idx + 2048], sw)

    tasks = ([("e", ch) for ch in range(E_CHUNKS)]
             + [("f", ch) for ch in range(F_CHUNKS)])

    def _fire(t, p):
        kind, ch = tasks[t]
        cps = []
        if kind == "e":
            blk = wid * E_CHUNKS + ch
            pltpu.sync_copy(ei0_hbm.at[blk], idx_v.at[p, 0])
            pltpu.sync_copy(ei1_hbm.at[blk], idx_v.at[p, 1])
            nrows = 2
        else:
            blk = wid * F_CHUNKS + ch
            pltpu.sync_copy(fa0_hbm.at[blk], idx_v.at[p, 0])
            pltpu.sync_copy(fa1_hbm.at[blk], idx_v.at[p, 1])
            pltpu.sync_copy(fa2_hbm.at[blk], idx_v.at[p, 2])
            nrows = 3
        for r in range(nrows):
            for a in range(4):
                cps.append(pltpu.async_copy(
                    packed_hbm.at[idx_v.at[p, r, a]],
                    rows_v.at[p, r, pl.ds(a * 128, 128)], sems.at[p]))
        return cps

    def _compute(t, p):
        kind, _ = tasks[t]
        if kind == "e":
            @plsc.parallel_loop(0, CHUNK, 1, unroll=4)
            def _ebody(e):
                h = jnp.minimum(rows_v[p, 0, e, 0:16], rows_v[p, 1, e, 0:16])
                s0 = rows_v[p, 0, e, 16:32]
                wm = jnp.maximum(s0, rows_v[p, 1, e, 16:32])
                zi = jnp.zeros((16,), jnp.int32)
                wv = wm.at[zi].get(mode="promise_in_bounds")
                bv = s0.at[zi + 1].get(mode="promise_in_bounds")
                _accum(h, -wv, bv)
        else:
            @plsc.parallel_loop(0, CHUNK, 1, unroll=4)
            def _fbody(e):
                h = jnp.minimum(
                    jnp.minimum(rows_v[p, 0, e, 0:16], rows_v[p, 1, e, 0:16]),
                    rows_v[p, 2, e, 0:16])
                s0 = rows_v[p, 0, e, 16:32]
                wm = jnp.maximum(jnp.maximum(s0, rows_v[p, 1, e, 16:32]),
                                 rows_v[p, 2, e, 16:32])
                zi = jnp.zeros((16,), jnp.int32)
                wv = wm.at[zi].get(mode="promise_in_bounds")
                bv = s0.at[zi + 1].get(mode="promise_in_bounds")
                _accum(h, wv, bv)

    pending = _fire(0, 0)
    for t in range(len(tasks)):
        nxt = _fire(t + 1, (t + 1) % 2) if t + 1 < len(tasks) else None
        for cp in pending:
            cp.wait()
        _compute(t, t % 2)
        pending = nxt

    ncp.wait()

    @plsc.parallel_loop(0, N_CHUNK, 1, unroll=4)
    def _nbody(e):
        h = node_v[e, 0:16]
        s0 = node_v[e, 16:32]
        zi = jnp.zeros((16,), jnp.int32)
        wv = s0.at[zi].get(mode="promise_in_bounds")
        bv = s0.at[zi + 1].get(mode="promise_in_bounds")
        _accum(h, wv, bv)

    def _stage(i, carry):
        stage_v[i, :] = hist_v[pl.ds(i * 16, 16)]
        return carry

    lax.fori_loop(0, 256, _stage, 0)
    pltpu.sync_copy(stage_v, out_hbm.at[wid])


_sc_call = pl.kernel(
    _sc_body,
    out_type=jax.ShapeDtypeStruct((NW, 256, 16), jnp.float32),
    mesh=plsc.VectorSubcoreMesh(core_axis_name="c", subcore_axis_name="s"),
    compiler_params=pltpu.CompilerParams(needs_layout_passes=False,
                                         use_tc_tiling_on_sc=False),
    scratch_types=[
        pltpu.VMEM((4096,), jnp.float32),
        pltpu.VMEM((256, 16), jnp.float32),
        pltpu.VMEM((2, 3, CHUNK, 32), jnp.float32),
        pltpu.VMEM((N_CHUNK, 32), jnp.float32),
        pltpu.VMEM((2, 3, 4, 128), jnp.int32),
        pltpu.SemaphoreType.DMA((2,)),
        pltpu.SemaphoreType.DMA,
    ],
)


@jax.jit
def kernel(x, edge_index, face, node_weights, batch, v):
    ei = edge_index.astype(jnp.int32)
    fa = face.astype(jnp.int32)

    xp = jnp.concatenate([x, jnp.zeros((N_PAD - N, D), jnp.float32)], axis=0)
    wp = jnp.concatenate([node_weights,
                          jnp.zeros((N_PAD - N,), jnp.float32)])[:, None]
    bp = jnp.concatenate([batch.astype(jnp.float32) * 256.0,
                          jnp.zeros((N_PAD - N,), jnp.float32)])[:, None]

    packed = pl.pallas_call(
        _pack_body,
        out_shape=jax.ShapeDtypeStruct((N_PAD, 32), jnp.float32),
    )(xp, v, wp, bp)

    epad = jnp.full((E_PAD - E,), N, jnp.int32)
    fpad = jnp.full((F_PAD - F,), N, jnp.int32)
    ei0 = jnp.concatenate([ei[0], epad]).reshape(NW * E_CHUNKS, 4, 128)
    ei1 = jnp.concatenate([ei[1], epad]).reshape(NW * E_CHUNKS, 4, 128)
    fa0 = jnp.concatenate([fa[0], fpad]).reshape(NW * F_CHUNKS, 4, 128)
    fa1 = jnp.concatenate([fa[1], fpad]).reshape(NW * F_CHUNKS, 4, 128)
    fa2 = jnp.concatenate([fa[2], fpad]).reshape(NW * F_CHUNKS, 4, 128)

    hist = _sc_call(packed, ei0, ei1, fa0, fa1, fa2)

    out2 = pl.pallas_call(
        _fin_body,
        out_shape=jax.ShapeDtypeStruct((128, 16), jnp.float32),
    )(hist)
    return out2.reshape(B, S, T)


# speedup vs baseline: 2.1925x; 2.1842x over previous
import functools

import jax
import jax.numpy as jnp
from jax import lax
from jax.experimental import pallas as pl
from jax.experimental.pallas import tpu as pltpu
from jax.experimental.pallas import tpu_sc as plsc

N = 10000
E = 160000
F = 20000
D = 3
T = 16
S = 16
R = 1.1
B = 8

DELTA = 2.0 * R / (S - 1)
NW = 32
N_PAD = 10240
E_PAD = 163840
F_PAD = 20480
E_CHUNK = 1024
E_CHUNKS = 5
F_CHUNK = 640
N_CHUNK = 320


def _pack_body(x_ref, v_ref, w_ref, b_ref, out_ref):
    xv = x_ref[...]
    vv = v_ref[...]
    nh = (xv[:, 0:1] * vv[0:1, :]
          + xv[:, 1:2] * vv[1:2, :]
          + xv[:, 2:3] * vv[2:3, :])
    out_ref[:, 0:16] = nh
    out_ref[:, 16:17] = w_ref[...]
    out_ref[:, 17:18] = b_ref[...]
    out_ref[:, 18:32] = jnp.zeros((N_PAD, 14), jnp.float32)


def _fin_body(hist_ref, out_ref):
    s = jnp.sum(hist_ref[...], axis=0)
    h2 = s[0:128, :]
    g2 = s[128:256, :]
    r = lax.broadcasted_iota(jnp.int32, (128, 128), 0)
    c = lax.broadcasted_iota(jnp.int32, (128, 128), 1)
    m = ((r >> 4) == (c >> 4)) & ((c & 15) < (r & 15))
    out_ref[...] = h2 + jnp.dot(m.astype(jnp.float32), g2,
                                preferred_element_type=jnp.float32)


def _sc_body(packed_hbm, ei0_hbm, ei1_hbm, fa0_hbm, fa1_hbm, fa2_hbm,
             out_hbm, hist_v, stage_v, r0_v, r1_v, r2_v, i0_v, i1_v,
             f0_v, f1_v, f2_v, sem):
    cid = lax.axis_index("c")
    sid = lax.axis_index("s")
    wid = sid * 2 + cid
    lane = lax.iota(jnp.int32, 16)

    zero16 = jnp.zeros((16,), jnp.float32)

    def _zero(i, carry):
        hist_v[pl.ds(i * 16, 16)] = zero16
        return carry

    lax.fori_loop(0, 256, _zero, 0)

    inv = 1.0 / DELTA
    c0 = R / DELTA + 0.5
    scale = -500.0 * DELTA

    def _accum(h, sw, bv):
        u = h * inv + c0
        jf = jnp.minimum(jnp.maximum(u, 0.0), 15.0)
        j = jf.astype(jnp.int32)
        jq = j.astype(jnp.float32)
        z = jq * scale + (h * 500.0 + 500.0 * R)
        z = jnp.minimum(z, 30.0)
        wsig = sw / (1.0 + jnp.exp(z))
        idx = bv.astype(jnp.int32) + j * 16 + lane
        plsc.addupdate_scatter(hist_v, [idx], wsig)
        plsc.addupdate_scatter(hist_v, [idx + 2048], sw)

    for ch in range(E_CHUNKS):
        blk = wid * E_CHUNKS + ch
        pltpu.sync_copy(ei0_hbm.at[blk], i0_v)
        pltpu.sync_copy(ei1_hbm.at[blk], i1_v)
        cps = []
        for a in range(8):
            cps.append(pltpu.async_copy(
                packed_hbm.at[i0_v.at[a]], r0_v.at[pl.ds(a * 128, 128)], sem))
            cps.append(pltpu.async_copy(
                packed_hbm.at[i1_v.at[a]], r1_v.at[pl.ds(a * 128, 128)], sem))
        for cp in cps:
            cp.wait()

        @plsc.parallel_loop(0, E_CHUNK, 1, unroll=4)
        def _ebody(e):
            h = jnp.minimum(r0_v[e, 0:16], r1_v[e, 0:16])
            s0 = r0_v[e, 16:32]
            s1 = r1_v[e, 16:32]
            wm = jnp.maximum(s0, s1)
            zi = jnp.zeros((16,), jnp.int32)
            wv = wm.at[zi].get(mode="promise_in_bounds")
            bv = s0.at[zi + 1].get(mode="promise_in_bounds")
            _accum(h, -wv, bv)

    pltpu.sync_copy(fa0_hbm.at[wid], f0_v)
    pltpu.sync_copy(fa1_hbm.at[wid], f1_v)
    pltpu.sync_copy(fa2_hbm.at[wid], f2_v)
    cps = []
    for a in range(5):
        cps.append(pltpu.async_copy(
            packed_hbm.at[f0_v.at[a]], r0_v.at[pl.ds(a * 128, 128)], sem))
        cps.append(pltpu.async_copy(
            packed_hbm.at[f1_v.at[a]], r1_v.at[pl.ds(a * 128, 128)], sem))
        cps.append(pltpu.async_copy(
            packed_hbm.at[f2_v.at[a]], r2_v.at[pl.ds(a * 128, 128)], sem))
    for cp in cps:
        cp.wait()

    @plsc.parallel_loop(0, F_CHUNK, 1, unroll=4)
    def _fbody(e):
        h = jnp.minimum(jnp.minimum(r0_v[e, 0:16], r1_v[e, 0:16]),
                        r2_v[e, 0:16])
        s0 = r0_v[e, 16:32]
        wm = jnp.maximum(jnp.maximum(s0, r1_v[e, 16:32]), r2_v[e, 16:32])
        zi = jnp.zeros((16,), jnp.int32)
        wv = wm.at[zi].get(mode="promise_in_bounds")
        bv = s0.at[zi + 1].get(mode="promise_in_bounds")
        _accum(h, wv, bv)

    nbase = pl.multiple_of(wid * N_CHUNK, N_CHUNK)
    pltpu.sync_copy(packed_hbm.at[pl.ds(nbase, N_CHUNK)],
                    r0_v.at[pl.ds(0, N_CHUNK)])

    @plsc.parallel_loop(0, N_CHUNK, 1, unroll=4)
    def _nbody(e):
        h = r0_v[e, 0:16]
        s0 = r0_v[e, 16:32]
        zi = jnp.zeros((16,), jnp.int32)
        wv = s0.at[zi].get(mode="promise_in_bounds")
        bv = s0.at[zi + 1].get(mode="promise_in_bounds")
        _accum(h, wv, bv)

    def _stage(i, carry):
        stage_v[i, :] = hist_v[pl.ds(i * 16, 16)]
        return carry

    lax.fori_loop(0, 256, _stage, 0)
    pltpu.sync_copy(stage_v, out_hbm.at[wid])


_sc_call = pl.kernel(
    _sc_body,
    out_type=jax.ShapeDtypeStruct((NW, 256, 16), jnp.float32),
    mesh=plsc.VectorSubcoreMesh(core_axis_name="c", subcore_axis_name="s"),
    compiler_params=pltpu.CompilerParams(needs_layout_passes=False,
                                         use_tc_tiling_on_sc=False),
    scratch_types=[
        pltpu.VMEM((4096,), jnp.float32),
        pltpu.VMEM((256, 16), jnp.float32),
        pltpu.VMEM((E_CHUNK, 32), jnp.float32),
        pltpu.VMEM((E_CHUNK, 32), jnp.float32),
        pltpu.VMEM((E_CHUNK, 32), jnp.float32),
        pltpu.VMEM((8, 128), jnp.int32),
        pltpu.VMEM((8, 128), jnp.int32),
        pltpu.VMEM((5, 128), jnp.int32),
        pltpu.VMEM((5, 128), jnp.int32),
        pltpu.VMEM((5, 128), jnp.int32),
        pltpu.SemaphoreType.DMA,
    ],
)


@jax.jit
def kernel(x, edge_index, face, node_weights, batch, v):
    ei = edge_index.astype(jnp.int32)
    fa = face.astype(jnp.int32)

    xp = jnp.concatenate([x, jnp.zeros((N_PAD - N, D), jnp.float32)], axis=0)
    wp = jnp.concatenate([node_weights,
                          jnp.zeros((N_PAD - N,), jnp.float32)])[:, None]
    bp = jnp.concatenate([batch.astype(jnp.float32) * 256.0,
                          jnp.zeros((N_PAD - N,), jnp.float32)])[:, None]

    packed = pl.pallas_call(
        _pack_body,
        out_shape=jax.ShapeDtypeStruct((N_PAD, 32), jnp.float32),
    )(xp, v, wp, bp)

    epad = jnp.full((E_PAD - E,), N, jnp.int32)
    fpad = jnp.full((F_PAD - F,), N, jnp.int32)
    ei0 = jnp.concatenate([ei[0], epad]).reshape(NW * E_CHUNKS, 8, 128)
    ei1 = jnp.concatenate([ei[1], epad]).reshape(NW * E_CHUNKS, 8, 128)
    fa0 = jnp.concatenate([fa[0], fpad]).reshape(NW, 5, 128)
    fa1 = jnp.concatenate([fa[1], fpad]).reshape(NW, 5, 128)
    fa2 = jnp.concatenate([fa[2], fpad]).reshape(NW, 5, 128)

    hist = _sc_call(packed, ei0, ei1, fa0, fa1, fa2)

    out2 = pl.pallas_call(
        _fin_body,
        out_shape=jax.ShapeDtypeStruct((128, 16), jnp.float32),
    )(hist)
    return out2.reshape(B, S, T)


# speedup vs baseline: 3.9315x; 1.7932x over previous
import functools

import jax
import jax.numpy as jnp
from jax import lax
from jax.experimental import pallas as pl
from jax.experimental.pallas import tpu as pltpu
from jax.experimental.pallas import tpu_sc as plsc

N = 10000
E = 160000
F = 20000
D = 3
T = 16
S = 16
R = 1.1
B = 8

DELTA = 2.0 * R / (S - 1)
NW = 32
N_PAD = 10240
E_CHUNK = 1000
E_CHUNKS = 5
F_CHUNK = 625
N_CHUNK = 320
E_SUB = 125
NH = 16


def _pack_body(x_ref, v_ref, w_ref, b_ref, out_ref):
    xv = x_ref[...]
    vv = v_ref[...]
    nh = (xv[:, 0:1] * vv[0:1, :]
          + xv[:, 1:2] * vv[1:2, :]
          + xv[:, 2:3] * vv[2:3, :])
    out_ref[0:N, 0:16] = nh
    out_ref[0:N, 16:17] = w_ref[...]
    out_ref[0:N, 17:18] = b_ref[...].astype(jnp.float32) * 256.0
    out_ref[0:N, 18:32] = jnp.zeros((N, 14), jnp.float32)
    out_ref[N:N_PAD, :] = jnp.zeros((N_PAD - N, 32), jnp.float32)


def _fin_body(hist_ref, out_ref):
    s = jnp.sum(hist_ref[...], axis=0)
    h2 = s[0:128, :]
    g2 = s[128:256, :]
    r = lax.broadcasted_iota(jnp.int32, (128, 128), 0)
    c = lax.broadcasted_iota(jnp.int32, (128, 128), 1)
    m = ((r >> 4) == (c >> 4)) & ((c & 15) < (r & 15))
    out_ref[...] = h2 + jnp.dot(m.astype(jnp.float32), g2,
                                preferred_element_type=jnp.float32)


def _sc_body(packed_hbm, ei0_hbm, ei1_hbm, fa0_hbm, fa1_hbm, fa2_hbm,
             out_hbm, hist_v, stage_v, r0_v, r1_v, r2_v, i0_v, i1_v,
             f0_v, f1_v, f2_v, sem):
    cid = lax.axis_index("c")
    sid = lax.axis_index("s")
    wid = sid * 2 + cid
    lane = lax.iota(jnp.int32, 16)

    zero16 = jnp.zeros((16,), jnp.float32)

    def _zero(i, carry):
        hist_v[pl.ds(i * 16, 16)] = zero16
        return carry

    lax.fori_loop(0, 256, _zero, 0)

    inv = 1.0 / DELTA
    c0 = R / DELTA + 0.5
    scale = -500.0 * DELTA

    def _accum(h, sw, bv):
        u = h * inv + c0
        jf = jnp.minimum(jnp.maximum(u, 0.0), 15.0)
        j = jf.astype(jnp.int32)
        jq = j.astype(jnp.float32)
        z = jq * scale + (h * 500.0 + 500.0 * R)
        z = jnp.minimum(z, 30.0)
        wsig = sw / (1.0 + jnp.exp(z))
        idx = bv.astype(jnp.int32) + j * 16 + lane
        plsc.addupdate_scatter(hist_v, [idx], wsig)
        plsc.addupdate_scatter(hist_v, [idx + 2048], sw)

    for ch in range(E_CHUNKS):
        blk = wid * E_CHUNKS + ch
        pltpu.sync_copy(ei0_hbm.at[blk], i0_v)
        pltpu.sync_copy(ei1_hbm.at[blk], i1_v)
        cps = []
        for a in range(8):
            cps.append(pltpu.async_copy(
                packed_hbm.at[i0_v.at[a]],
                r0_v.at[pl.ds(a * E_SUB, E_SUB)], sem))
            cps.append(pltpu.async_copy(
                packed_hbm.at[i1_v.at[a]],
                r1_v.at[pl.ds(a * E_SUB, E_SUB)], sem))
        for cp in cps:
            cp.wait()

        @plsc.parallel_loop(0, E_CHUNK, 1, unroll=4)
        def _ebody(e):
            h = jnp.minimum(r0_v[e, 0:16], r1_v[e, 0:16])
            s0 = r0_v[e, 16:32]
            s1 = r1_v[e, 16:32]
            wm = jnp.maximum(s0, s1)
            zi = jnp.zeros((16,), jnp.int32)
            wv = wm.at[zi].get(mode="promise_in_bounds")
            bv = s0.at[zi + 1].get(mode="promise_in_bounds")
            _accum(h, -wv, bv)

    pltpu.sync_copy(fa0_hbm.at[wid], f0_v)
    pltpu.sync_copy(fa1_hbm.at[wid], f1_v)
    pltpu.sync_copy(fa2_hbm.at[wid], f2_v)
    cps = []
    for a in range(5):
        cps.append(pltpu.async_copy(
            packed_hbm.at[f0_v.at[a]],
            r0_v.at[pl.ds(a * E_SUB, E_SUB)], sem))
        cps.append(pltpu.async_copy(
            packed_hbm.at[f1_v.at[a]],
            r1_v.at[pl.ds(a * E_SUB, E_SUB)], sem))
        cps.append(pltpu.async_copy(
            packed_hbm.at[f2_v.at[a]],
            r2_v.at[pl.ds(a * E_SUB, E_SUB)], sem))
    for cp in cps:
        cp.wait()

    @plsc.parallel_loop(0, F_CHUNK, 1, unroll=4)
    def _fbody(e):
        h = jnp.minimum(jnp.minimum(r0_v[e, 0:16], r1_v[e, 0:16]),
                        r2_v[e, 0:16])
        s0 = r0_v[e, 16:32]
        wm = jnp.maximum(jnp.maximum(s0, r1_v[e, 16:32]), r2_v[e, 16:32])
        zi = jnp.zeros((16,), jnp.int32)
        wv = wm.at[zi].get(mode="promise_in_bounds")
        bv = s0.at[zi + 1].get(mode="promise_in_bounds")
        _accum(h, wv, bv)

    nbase = pl.multiple_of(wid * N_CHUNK, N_CHUNK)
    pltpu.sync_copy(packed_hbm.at[pl.ds(nbase, N_CHUNK)],
                    r0_v.at[pl.ds(0, N_CHUNK)])

    @plsc.parallel_loop(0, N_CHUNK, 1, unroll=4)
    def _nbody(e):
        h = r0_v[e, 0:16]
        s0 = r0_v[e, 16:32]
        zi = jnp.zeros((16,), jnp.int32)
        wv = s0.at[zi].get(mode="promise_in_bounds")
        bv = s0.at[zi + 1].get(mode="promise_in_bounds")
        _accum(h, wv, bv)

    def _stage(i, carry):
        stage_v[i, :] = hist_v[pl.ds(i * 16, 16)]
        return carry

    lax.fori_loop(0, 256, _stage, 0)
    pltpu.sync_copy(stage_v, out_hbm.at[wid])


_sc_call = pl.kernel(
    _sc_body,
    out_type=jax.ShapeDtypeStruct((NW, 256, 16), jnp.float32),
    mesh=plsc.VectorSubcoreMesh(core_axis_name="c", subcore_axis_name="s"),
    compiler_params=pltpu.CompilerParams(needs_layout_passes=False,
                                         use_tc_tiling_on_sc=False),
    scratch_types=[
        pltpu.VMEM((4096,), jnp.float32),
        pltpu.VMEM((256, 16), jnp.float32),
        pltpu.VMEM((E_CHUNK, 32), jnp.float32),
        pltpu.VMEM((E_CHUNK, 32), jnp.float32),
        pltpu.VMEM((E_CHUNK, 32), jnp.float32),
        pltpu.VMEM((8, E_SUB), jnp.int32),
        pltpu.VMEM((8, E_SUB), jnp.int32),
        pltpu.VMEM((5, E_SUB), jnp.int32),
        pltpu.VMEM((5, E_SUB), jnp.int32),
        pltpu.VMEM((5, E_SUB), jnp.int32),
        pltpu.SemaphoreType.DMA,
    ],
)


@jax.jit
def kernel(x, edge_index, face, node_weights, batch, v):
    ei = edge_index.astype(jnp.int32)
    fa = face.astype(jnp.int32)

    packed = pl.pallas_call(
        _pack_body,
        out_shape=jax.ShapeDtypeStruct((N_PAD, 32), jnp.float32),
    )(x, v, node_weights[:, None], batch.astype(jnp.int32)[:, None])

    ei0 = ei[0].reshape(NW * E_CHUNKS, 8, E_SUB)
    ei1 = ei[1].reshape(NW * E_CHUNKS, 8, E_SUB)
    fa0 = fa[0].reshape(NW, 5, E_SUB)
    fa1 = fa[1].reshape(NW, 5, E_SUB)
    fa2 = fa[2].reshape(NW, 5, E_SUB)

    hist = _sc_call(packed, ei0, ei1, fa0, fa1, fa2)

    out2 = pl.pallas_call(
        _fin_body,
        out_shape=jax.ShapeDtypeStruct((128, 16), jnp.float32),
    )(hist)
    return out2.reshape(B, S, T)


# speedup vs baseline: 4.1078x; 1.0449x over previous
import functools

import jax
import jax.numpy as jnp
from jax import lax
from jax.experimental import pallas as pl
from jax.experimental.pallas import tpu as pltpu
from jax.experimental.pallas import tpu_sc as plsc

N = 10000
E = 160000
F = 20000
D = 3
T = 16
S = 16
R = 1.1
B = 8

DELTA = 2.0 * R / (S - 1)
NW = 32
N_PAD = 10240
E_CHUNK = 1000
E_CHUNKS = 5
F_CHUNK = 625
N_CHUNK = 320
E_SUB = 125
NH = 16


def _pack_body(x_ref, v_ref, w_ref, b_ref, out_ref):
    nh = jnp.dot(x_ref[...], v_ref[...],
                 preferred_element_type=jnp.float32)
    row = jnp.concatenate(
        [nh, w_ref[...], b_ref[...].astype(jnp.float32) * 256.0,
         jnp.zeros((N, 14), jnp.float32)], axis=1)
    out_ref[0:N, :] = row
    out_ref[N:N_PAD, :] = jnp.zeros((N_PAD - N, 32), jnp.float32)


def _fin_body(hist_ref, out_ref):
    s = jnp.sum(hist_ref[...], axis=0)
    h2 = s[0:128, :]
    g2 = s[128:256, :]
    r = lax.broadcasted_iota(jnp.int32, (128, 128), 0)
    c = lax.broadcasted_iota(jnp.int32, (128, 128), 1)
    m = ((r >> 4) == (c >> 4)) & ((c & 15) < (r & 15))
    out_ref[...] = h2 + jnp.dot(m.astype(jnp.float32), g2,
                                preferred_element_type=jnp.float32)


def _sc_body(packed_hbm, ei0_hbm, ei1_hbm, fa0_hbm, fa1_hbm, fa2_hbm,
             out_hbm, hist_v, stage_v, r0_v, r1_v, r2_v, i0_v, i1_v,
             f0_v, f1_v, f2_v, sem):
    cid = lax.axis_index("c")
    sid = lax.axis_index("s")
    wid = sid * 2 + cid
    lane = lax.iota(jnp.int32, 16)

    zero16 = jnp.zeros((16,), jnp.float32)

    def _zero(i, carry):
        hist_v[pl.ds(i * 16, 16)] = zero16
        return carry

    lax.fori_loop(0, 256, _zero, 0)

    inv = 1.0 / DELTA
    c0 = R / DELTA + 0.5
    scale = -500.0 * DELTA

    def _accum(h, sw, bv):
        u = h * inv + c0
        jf = jnp.minimum(jnp.maximum(u, 0.0), 15.0)
        j = jf.astype(jnp.int32)
        jq = j.astype(jnp.float32)
        z = jq * scale + (h * 500.0 + 500.0 * R)
        z = jnp.minimum(z, 30.0)
        wsig = sw / (1.0 + jnp.exp(z))
        idx = bv.astype(jnp.int32) + j * 16 + lane
        plsc.addupdate_scatter(hist_v, [idx], wsig)
        plsc.addupdate_scatter(hist_v, [idx + 2048], sw)

    for ch in range(E_CHUNKS):
        blk = wid * E_CHUNKS + ch
        pltpu.sync_copy(ei0_hbm.at[blk], i0_v)
        pltpu.sync_copy(ei1_hbm.at[blk], i1_v)
        cps = []
        for a in range(8):
            cps.append(pltpu.async_copy(
                packed_hbm.at[i0_v.at[a]],
                r0_v.at[pl.ds(a * E_SUB, E_SUB)], sem))
            cps.append(pltpu.async_copy(
                packed_hbm.at[i1_v.at[a]],
                r1_v.at[pl.ds(a * E_SUB, E_SUB)], sem))
        for cp in cps:
            cp.wait()

        @plsc.parallel_loop(0, E_CHUNK, 1, unroll=8)
        def _ebody(e):
            h = jnp.minimum(r0_v[e, 0:16], r1_v[e, 0:16])
            s0 = r0_v[e, 16:32]
            s1 = r1_v[e, 16:32]
            wm = jnp.maximum(s0, s1)
            zi = jnp.zeros((16,), jnp.int32)
            wv = wm.at[zi].get(mode="promise_in_bounds")
            bv = s0.at[zi + 1].get(mode="promise_in_bounds")
            _accum(h, -wv, bv)

    pltpu.sync_copy(fa0_hbm.at[wid], f0_v)
    pltpu.sync_copy(fa1_hbm.at[wid], f1_v)
    pltpu.sync_copy(fa2_hbm.at[wid], f2_v)
    cps = []
    for a in range(5):
        cps.append(pltpu.async_copy(
            packed_hbm.at[f0_v.at[a]],
            r0_v.at[pl.ds(a * E_SUB, E_SUB)], sem))
        cps.append(pltpu.async_copy(
            packed_hbm.at[f1_v.at[a]],
            r1_v.at[pl.ds(a * E_SUB, E_SUB)], sem))
        cps.append(pltpu.async_copy(
            packed_hbm.at[f2_v.at[a]],
            r2_v.at[pl.ds(a * E_SUB, E_SUB)], sem))
    for cp in cps:
        cp.wait()

    @plsc.parallel_loop(0, F_CHUNK, 1, unroll=8)
    def _fbody(e):
        h = jnp.minimum(jnp.minimum(r0_v[e, 0:16], r1_v[e, 0:16]),
                        r2_v[e, 0:16])
        s0 = r0_v[e, 16:32]
        wm = jnp.maximum(jnp.maximum(s0, r1_v[e, 16:32]), r2_v[e, 16:32])
        zi = jnp.zeros((16,), jnp.int32)
        wv = wm.at[zi].get(mode="promise_in_bounds")
        bv = s0.at[zi + 1].get(mode="promise_in_bounds")
        _accum(h, wv, bv)

    nbase = pl.multiple_of(wid * N_CHUNK, N_CHUNK)
    pltpu.sync_copy(packed_hbm.at[pl.ds(nbase, N_CHUNK)],
                    r0_v.at[pl.ds(0, N_CHUNK)])

    @plsc.parallel_loop(0, N_CHUNK, 1, unroll=8)
    def _nbody(e):
        h = r0_v[e, 0:16]
        s0 = r0_v[e, 16:32]
        zi = jnp.zeros((16,), jnp.int32)
        wv = s0.at[zi].get(mode="promise_in_bounds")
        bv = s0.at[zi + 1].get(mode="promise_in_bounds")
        _accum(h, wv, bv)

    def _stage(i, carry):
        stage_v[i, :] = hist_v[pl.ds(i * 16, 16)]
        return carry

    lax.fori_loop(0, 256, _stage, 0)
    pltpu.sync_copy(stage_v, out_hbm.at[wid])


_sc_call = pl.kernel(
    _sc_body,
    out_type=jax.ShapeDtypeStruct((NW, 256, 16), jnp.float32),
    mesh=plsc.VectorSubcoreMesh(core_axis_name="c", subcore_axis_name="s"),
    compiler_params=pltpu.CompilerParams(needs_layout_passes=False,
                                         use_tc_tiling_on_sc=False),
    scratch_types=[
        pltpu.VMEM((4096,), jnp.float32),
        pltpu.VMEM((256, 16), jnp.float32),
        pltpu.VMEM((E_CHUNK, 32), jnp.float32),
        pltpu.VMEM((E_CHUNK, 32), jnp.float32),
        pltpu.VMEM((E_CHUNK, 32), jnp.float32),
        pltpu.VMEM((8, E_SUB), jnp.int32),
        pltpu.VMEM((8, E_SUB), jnp.int32),
        pltpu.VMEM((5, E_SUB), jnp.int32),
        pltpu.VMEM((5, E_SUB), jnp.int32),
        pltpu.VMEM((5, E_SUB), jnp.int32),
        pltpu.SemaphoreType.DMA,
    ],
)


@jax.jit
def kernel(x, edge_index, face, node_weights, batch, v):
    ei = edge_index.astype(jnp.int32)
    fa = face.astype(jnp.int32)

    packed = pl.pallas_call(
        _pack_body,
        out_shape=jax.ShapeDtypeStruct((N_PAD, 32), jnp.float32),
    )(x, v, node_weights[:, None], batch.astype(jnp.int32)[:, None])

    ei0 = ei[0].reshape(NW * E_CHUNKS, 8, E_SUB)
    ei1 = ei[1].reshape(NW * E_CHUNKS, 8, E_SUB)
    fa0 = fa[0].reshape(NW, 5, E_SUB)
    fa1 = fa[1].reshape(NW, 5, E_SUB)
    fa2 = fa[2].reshape(NW, 5, E_SUB)

    hist = _sc_call(packed, ei0, ei1, fa0, fa1, fa2)

    out2 = pl.pallas_call(
        _fin_body,
        out_shape=jax.ShapeDtypeStruct((128, 16), jnp.float32),
    )(hist)
    return out2.reshape(B, S, T)


# speedup vs baseline: 4.3697x; 1.0637x over previous
import functools

import jax
import jax.numpy as jnp
from jax import lax
from jax.experimental import pallas as pl
from jax.experimental.pallas import tpu as pltpu
from jax.experimental.pallas import tpu_sc as plsc

N = 10000
E = 160000
F = 20000
D = 3
T = 16
S = 16
R = 1.1
B = 8

DELTA = 2.0 * R / (S - 1)
NW = 32
N_PAD = 10240
E_CHUNK = 1000
E_CHUNKS = 5
F_CHUNK = 625
N_CHUNK = 320
E_SUB = 125
NH = 16


def _pack_body(x_ref, v_ref, w_ref, b_ref, out_ref):
    nh = jnp.dot(x_ref[...], v_ref[...],
                 preferred_element_type=jnp.float32)
    row = jnp.concatenate(
        [nh, w_ref[...], b_ref[...].astype(jnp.float32) * 256.0,
         jnp.zeros((N, 14), jnp.float32)], axis=1)
    out_ref[0:N, :] = row
    out_ref[N:N_PAD, :] = jnp.zeros((N_PAD - N, 32), jnp.float32)


def _fin_body(hist_ref, out_ref):
    s = jnp.sum(hist_ref[...], axis=0)
    h2 = s[0:128, :]
    g2 = s[128:256, :]
    r = lax.broadcasted_iota(jnp.int32, (128, 128), 0)
    c = lax.broadcasted_iota(jnp.int32, (128, 128), 1)
    m = ((r >> 4) == (c >> 4)) & ((c & 15) < (r & 15))
    out_ref[...] = h2 + jnp.dot(m.astype(jnp.float32), g2,
                                preferred_element_type=jnp.float32)


def _sc_body(packed_hbm, ei_hbm, fa_hbm,
             out_hbm, hist_v, stage_v, r0_v, r1_v, r2_v, i0_v, i1_v,
             f0_v, f1_v, f2_v, sem):
    cid = lax.axis_index("c")
    sid = lax.axis_index("s")
    wid = sid * 2 + cid
    lane = lax.iota(jnp.int32, 16)

    zero16 = jnp.zeros((16,), jnp.float32)

    def _zero(i, carry):
        hist_v[pl.ds(i * 16, 16)] = zero16
        return carry

    lax.fori_loop(0, 256, _zero, 0)

    inv = 1.0 / DELTA
    c0 = R / DELTA + 0.5
    scale = -500.0 * DELTA

    def _accum(h, sw, bv):
        u = h * inv + c0
        jf = jnp.minimum(jnp.maximum(u, 0.0), 15.0)
        j = jf.astype(jnp.int32)
        jq = j.astype(jnp.float32)
        z = jq * scale + (h * 500.0 + 500.0 * R)
        z = jnp.minimum(z, 30.0)
        wsig = sw / (1.0 + jnp.exp(z))
        idx = bv.astype(jnp.int32) + j * 16 + lane
        plsc.addupdate_scatter(hist_v, [idx], wsig)
        plsc.addupdate_scatter(hist_v, [idx + 2048], sw)

    for ch in range(E_CHUNKS):
        blk = wid * E_CHUNKS + ch
        pltpu.sync_copy(ei_hbm.at[blk], i0_v)
        pltpu.sync_copy(ei_hbm.at[blk + NW * E_CHUNKS], i1_v)
        cps = []
        for a in range(8):
            cps.append(pltpu.async_copy(
                packed_hbm.at[i0_v.at[a]],
                r0_v.at[pl.ds(a * E_SUB, E_SUB)], sem))
            cps.append(pltpu.async_copy(
                packed_hbm.at[i1_v.at[a]],
                r1_v.at[pl.ds(a * E_SUB, E_SUB)], sem))
        for cp in cps:
            cp.wait()

        @plsc.parallel_loop(0, E_CHUNK, 1, unroll=8)
        def _ebody(e):
            h = jnp.minimum(r0_v[e, 0:16], r1_v[e, 0:16])
            s0 = r0_v[e, 16:32]
            s1 = r1_v[e, 16:32]
            wm = jnp.maximum(s0, s1)
            zi = jnp.zeros((16,), jnp.int32)
            wv = wm.at[zi].get(mode="promise_in_bounds")
            bv = s0.at[zi + 1].get(mode="promise_in_bounds")
            _accum(h, -wv, bv)

    pltpu.sync_copy(fa_hbm.at[wid], f0_v)
    pltpu.sync_copy(fa_hbm.at[wid + NW], f1_v)
    pltpu.sync_copy(fa_hbm.at[wid + 2 * NW], f2_v)
    cps = []
    for a in range(5):
        cps.append(pltpu.async_copy(
            packed_hbm.at[f0_v.at[a]],
            r0_v.at[pl.ds(a * E_SUB, E_SUB)], sem))
        cps.append(pltpu.async_copy(
            packed_hbm.at[f1_v.at[a]],
            r1_v.at[pl.ds(a * E_SUB, E_SUB)], sem))
        cps.append(pltpu.async_copy(
            packed_hbm.at[f2_v.at[a]],
            r2_v.at[pl.ds(a * E_SUB, E_SUB)], sem))
    for cp in cps:
        cp.wait()

    @plsc.parallel_loop(0, F_CHUNK, 1, unroll=8)
    def _fbody(e):
        h = jnp.minimum(jnp.minimum(r0_v[e, 0:16], r1_v[e, 0:16]),
                        r2_v[e, 0:16])
        s0 = r0_v[e, 16:32]
        wm = jnp.maximum(jnp.maximum(s0, r1_v[e, 16:32]), r2_v[e, 16:32])
        zi = jnp.zeros((16,), jnp.int32)
        wv = wm.at[zi].get(mode="promise_in_bounds")
        bv = s0.at[zi + 1].get(mode="promise_in_bounds")
        _accum(h, wv, bv)

    nbase = pl.multiple_of(wid * N_CHUNK, N_CHUNK)
    pltpu.sync_copy(packed_hbm.at[pl.ds(nbase, N_CHUNK)],
                    r0_v.at[pl.ds(0, N_CHUNK)])

    @plsc.parallel_loop(0, N_CHUNK, 1, unroll=8)
    def _nbody(e):
        h = r0_v[e, 0:16]
        s0 = r0_v[e, 16:32]
        zi = jnp.zeros((16,), jnp.int32)
        wv = s0.at[zi].get(mode="promise_in_bounds")
        bv = s0.at[zi + 1].get(mode="promise_in_bounds")
        _accum(h, wv, bv)

    def _stage(i, carry):
        stage_v[i, :] = hist_v[pl.ds(i * 16, 16)]
        return carry

    lax.fori_loop(0, 256, _stage, 0)
    pltpu.sync_copy(stage_v, out_hbm.at[wid])


_sc_call = pl.kernel(
    _sc_body,
    out_type=jax.ShapeDtypeStruct((NW, 256, 16), jnp.float32),
    mesh=plsc.VectorSubcoreMesh(core_axis_name="c", subcore_axis_name="s"),
    compiler_params=pltpu.CompilerParams(needs_layout_passes=False,
                                         use_tc_tiling_on_sc=False),
    scratch_types=[
        pltpu.VMEM((4096,), jnp.float32),
        pltpu.VMEM((256, 16), jnp.float32),
        pltpu.VMEM((E_CHUNK, 32), jnp.float32),
        pltpu.VMEM((E_CHUNK, 32), jnp.float32),
        pltpu.VMEM((E_CHUNK, 32), jnp.float32),
        pltpu.VMEM((8, E_SUB), jnp.int32),
        pltpu.VMEM((8, E_SUB), jnp.int32),
        pltpu.VMEM((5, E_SUB), jnp.int32),
        pltpu.VMEM((5, E_SUB), jnp.int32),
        pltpu.VMEM((5, E_SUB), jnp.int32),
        pltpu.SemaphoreType.DMA,
    ],
)


@jax.jit
def kernel(x, edge_index, face, node_weights, batch, v):
    packed = pl.pallas_call(
        _pack_body,
        out_shape=jax.ShapeDtypeStruct((N_PAD, 32), jnp.float32),
    )(x, v, node_weights[:, None], batch.astype(jnp.int32)[:, None])

    ei = edge_index.astype(jnp.int32).reshape(2 * NW * E_CHUNKS, 8, E_SUB)
    fa = face.astype(jnp.int32).reshape(3 * NW, 5, E_SUB)

    hist = _sc_call(packed, ei, fa)

    out2 = pl.pallas_call(
        _fin_body,
        out_shape=jax.ShapeDtypeStruct((128, 16), jnp.float32),
    )(hist)
    return out2.reshape(B, S, T)


# speedup vs baseline: 4.7925x; 1.0968x over previous
import functools

import jax
import jax.numpy as jnp
from jax import lax
from jax.experimental import pallas as pl
from jax.experimental.pallas import tpu as pltpu
from jax.experimental.pallas import tpu_sc as plsc

N = 10000
E = 160000
F = 20000
D = 3
T = 16
S = 16
R = 1.1
B = 8

DELTA = 2.0 * R / (S - 1)
NW = 32
N_PAD = 10240
E_CHUNK = 625
E_CHUNKS = 8
F_CHUNK = 625
N_CHUNK = 320
E_SUB = 125


def _pack_body(x_ref, v_ref, w_ref, b_ref, out_ref):
    nh = jnp.dot(x_ref[...], v_ref[...],
                 preferred_element_type=jnp.float32)
    row = jnp.concatenate(
        [nh, w_ref[...], b_ref[...].astype(jnp.float32) * 256.0,
         jnp.zeros((N, 14), jnp.float32)], axis=1)
    out_ref[0:N, :] = row
    out_ref[N:N_PAD, :] = jnp.zeros((N_PAD - N, 32), jnp.float32)


def _fin_body(hist_ref, out_ref):
    s = jnp.sum(hist_ref[...], axis=0)
    h2 = s[0:128, :]
    g2 = s[128:256, :]
    r = lax.broadcasted_iota(jnp.int32, (128, 128), 0)
    c = lax.broadcasted_iota(jnp.int32, (128, 128), 1)
    m = ((r >> 4) == (c >> 4)) & ((c & 15) < (r & 15))
    out_ref[...] = h2 + jnp.dot(m.astype(jnp.float32), g2,
                                preferred_element_type=jnp.float32)


def _sc_body(packed_hbm, ei_hbm, fa_hbm,
             out_hbm, hist_v, stage_v, r0a_v, r1a_v, r0b_v, r1b_v,
             node_v, i0_v, i1_v, f0_v, f1_v, f2_v, sem_i, sem_a, sem_b):
    cid = lax.axis_index("c")
    sid = lax.axis_index("s")
    wid = sid * 2 + cid
    lane = lax.iota(jnp.int32, 16)

    eibase = pl.multiple_of(wid * E_CHUNKS, 8)
    pre = [
        pltpu.async_copy(ei_hbm.at[pl.ds(eibase, E_CHUNKS)], i0_v, sem_i),
        pltpu.async_copy(ei_hbm.at[pl.ds(eibase + NW * E_CHUNKS, E_CHUNKS)],
                         i1_v, sem_i),
        pltpu.async_copy(fa_hbm.at[wid], f0_v, sem_i),
        pltpu.async_copy(fa_hbm.at[wid + NW], f1_v, sem_i),
        pltpu.async_copy(fa_hbm.at[wid + 2 * NW], f2_v, sem_i),
        pltpu.async_copy(
            packed_hbm.at[pl.ds(pl.multiple_of(wid * N_CHUNK, N_CHUNK),
                                N_CHUNK)], node_v, sem_i),
    ]

    zero16 = jnp.zeros((16,), jnp.float32)

    def _zero(i, carry):
        hist_v[pl.ds(i * 16, 16)] = zero16
        return carry

    lax.fori_loop(0, 256, _zero, 0)
    for cp in pre:
        cp.wait()

    inv = 1.0 / DELTA
    c0 = R / DELTA + 0.5
    scale = -500.0 * DELTA

    def _accum(h, sw, bv):
        u = h * inv + c0
        jf = jnp.minimum(jnp.maximum(u, 0.0), 15.0)
        j = jf.astype(jnp.int32)
        jq = j.astype(jnp.float32)
        z = jq * scale + (h * 500.0 + 500.0 * R)
        z = jnp.minimum(z, 30.0)
        wsig = sw / (1.0 + jnp.exp(z))
        idx = bv.astype(jnp.int32) + j * 16 + lane
        plsc.addupdate_scatter(hist_v, [idx], wsig)
        plsc.addupdate_scatter(hist_v, [idx + 2048], sw)

    def _fire(ch, r0, r1, sem):
        cps = []
        for a in range(5):
            cps.append(pltpu.async_copy(
                packed_hbm.at[i0_v.at[ch, a]],
                r0.at[pl.ds(a * E_SUB, E_SUB)], sem))
            cps.append(pltpu.async_copy(
                packed_hbm.at[i1_v.at[ch, a]],
                r1.at[pl.ds(a * E_SUB, E_SUB)], sem))
        return cps

    def _edge_compute(r0, r1):
        @plsc.parallel_loop(0, E_CHUNK, 1, unroll=8)
        def _ebody(e):
            h = jnp.minimum(r0[e, 0:16], r1[e, 0:16])
            s0 = r0[e, 16:32]
            wm = jnp.maximum(s0, r1[e, 16:32])
            zi = jnp.zeros((16,), jnp.int32)
            wv = wm.at[zi].get(mode="promise_in_bounds")
            bv = s0.at[zi + 1].get(mode="promise_in_bounds")
            _accum(h, -wv, bv)

    def _pair(g, carry):
        ca = _fire(2 * g, r0a_v, r1a_v, sem_a)
        cb = _fire(2 * g + 1, r0b_v, r1b_v, sem_b)
        for cp in ca:
            cp.wait()
        _edge_compute(r0a_v, r1a_v)
        for cp in cb:
            cp.wait()
        _edge_compute(r0b_v, r1b_v)
        return carry

    lax.fori_loop(0, E_CHUNKS // 2, _pair, 0)

    cps = []
    for a in range(5):
        for fv, rv in ((f0_v, r0a_v), (f1_v, r1a_v), (f2_v, r0b_v)):
            cps.append(pltpu.async_copy(
                packed_hbm.at[fv.at[a]],
                rv.at[pl.ds(a * E_SUB, E_SUB)], sem_a))
    for cp in cps:
        cp.wait()

    @plsc.parallel_loop(0, F_CHUNK, 1, unroll=8)
    def _fbody(e):
        h = jnp.minimum(jnp.minimum(r0a_v[e, 0:16], r1a_v[e, 0:16]),
                        r0b_v[e, 0:16])
        s0 = r0a_v[e, 16:32]
        wm = jnp.maximum(jnp.maximum(s0, r1a_v[e, 16:32]), r0b_v[e, 16:32])
        zi = jnp.zeros((16,), jnp.int32)
        wv = wm.at[zi].get(mode="promise_in_bounds")
        bv = s0.at[zi + 1].get(mode="promise_in_bounds")
        _accum(h, wv, bv)

    @plsc.parallel_loop(0, N_CHUNK, 1, unroll=8)
    def _nbody(e):
        h = node_v[e, 0:16]
        s0 = node_v[e, 16:32]
        zi = jnp.zeros((16,), jnp.int32)
        wv = s0.at[zi].get(mode="promise_in_bounds")
        bv = s0.at[zi + 1].get(mode="promise_in_bounds")
        _accum(h, wv, bv)

    def _stage(i, carry):
        stage_v[i, :] = hist_v[pl.ds(i * 16, 16)]
        return carry

    lax.fori_loop(0, 256, _stage, 0)
    pltpu.sync_copy(stage_v, out_hbm.at[wid])


_sc_call = pl.kernel(
    _sc_body,
    out_type=jax.ShapeDtypeStruct((NW, 256, 16), jnp.float32),
    mesh=plsc.VectorSubcoreMesh(core_axis_name="c", subcore_axis_name="s"),
    compiler_params=pltpu.CompilerParams(needs_layout_passes=False,
                                         use_tc_tiling_on_sc=False),
    scratch_types=[
        pltpu.VMEM((4096,), jnp.float32),
        pltpu.VMEM((256, 16), jnp.float32),
        pltpu.VMEM((E_CHUNK, 32), jnp.float32),
        pltpu.VMEM((E_CHUNK, 32), jnp.float32),
        pltpu.VMEM((E_CHUNK, 32), jnp.float32),
        pltpu.VMEM((E_CHUNK, 32), jnp.float32),
        pltpu.VMEM((N_CHUNK, 32), jnp.float32),
        pltpu.VMEM((E_CHUNKS, 5, E_SUB), jnp.int32),
        pltpu.VMEM((E_CHUNKS, 5, E_SUB), jnp.int32),
        pltpu.VMEM((5, E_SUB), jnp.int32),
        pltpu.VMEM((5, E_SUB), jnp.int32),
        pltpu.VMEM((5, E_SUB), jnp.int32),
        pltpu.SemaphoreType.DMA,
        pltpu.SemaphoreType.DMA,
        pltpu.SemaphoreType.DMA,
    ],
)


@jax.jit
def kernel(x, edge_index, face, node_weights, batch, v):
    packed = pl.pallas_call(
        _pack_body,
        out_shape=jax.ShapeDtypeStruct((N_PAD, 32), jnp.float32),
    )(x, v, node_weights[:, None], batch.astype(jnp.int32)[:, None])

    ei = edge_index.astype(jnp.int32).reshape(2 * NW * E_CHUNKS, 5, E_SUB)
    fa = face.astype(jnp.int32).reshape(3 * NW, 5, E_SUB)

    hist = _sc_call(packed, ei, fa)

    out2 = pl.pallas_call(
        _fin_body,
        out_shape=jax.ShapeDtypeStruct((128, 16), jnp.float32),
    )(hist)
    return out2.reshape(B, S, T)


# speedup vs baseline: 5.4146x; 1.1298x over previous
import functools

import jax
import jax.numpy as jnp
from jax import lax
from jax.experimental import pallas as pl
from jax.experimental.pallas import tpu as pltpu
from jax.experimental.pallas import tpu_sc as plsc

N = 10000
E = 160000
F = 20000
D = 3
T = 16
S = 16
R = 1.1
B = 8

DELTA = 2.0 * R / (S - 1)
NW = 32
N_PAD = 10240
E_CHUNK = 625
E_CHUNKS = 8
F_CHUNK = 625
N_CHUNK = 320
E_SUB = 125


def _pack_body(x_ref, v_ref, w_ref, b_ref, out_ref):
    nh = jnp.dot(x_ref[...], v_ref[...],
                 preferred_element_type=jnp.float32)
    row = jnp.concatenate(
        [nh, w_ref[...], b_ref[...].astype(jnp.float32) * 256.0,
         jnp.zeros((N, 14), jnp.float32)], axis=1)
    out_ref[0:N, :] = row
    out_ref[N:N_PAD, :] = jnp.zeros((N_PAD - N, 32), jnp.float32)


def _fin_body(hist_ref, out_ref):
    s = jnp.sum(hist_ref[...], axis=0)
    h2 = s[0:128, :]
    g2 = s[128:256, :]
    r = lax.broadcasted_iota(jnp.int32, (128, 128), 0)
    c = lax.broadcasted_iota(jnp.int32, (128, 128), 1)
    m = ((r >> 4) == (c >> 4)) & ((c & 15) < (r & 15))
    out_ref[...] = h2 + jnp.dot(m.astype(jnp.float32), g2,
                                preferred_element_type=jnp.float32)


def _sc_body(packed_hbm, ei_hbm, fa_hbm,
             out_hbm, hist_v, stage_v, r0a_v, r1a_v, r0b_v, r1b_v,
             node_v, i0_v, i1_v, f0_v, f1_v, f2_v, sem_i, sem_a, sem_b):
    cid = lax.axis_index("c")
    sid = lax.axis_index("s")
    wid = sid * 2 + cid
    lane = lax.iota(jnp.int32, 16)

    eibase = pl.multiple_of(wid * E_CHUNKS, 8)
    pre = [
        pltpu.async_copy(ei_hbm.at[pl.ds(eibase, E_CHUNKS)], i0_v, sem_i),
        pltpu.async_copy(ei_hbm.at[pl.ds(eibase + NW * E_CHUNKS, E_CHUNKS)],
                         i1_v, sem_i),
        pltpu.async_copy(fa_hbm.at[wid], f0_v, sem_i),
        pltpu.async_copy(fa_hbm.at[wid + NW], f1_v, sem_i),
        pltpu.async_copy(fa_hbm.at[wid + 2 * NW], f2_v, sem_i),
        pltpu.async_copy(
            packed_hbm.at[pl.ds(pl.multiple_of(wid * N_CHUNK, N_CHUNK),
                                N_CHUNK)], node_v, sem_i),
    ]

    zero16 = jnp.zeros((16,), jnp.float32)

    def _zero(i, carry):
        hist_v[pl.ds(i * 16, 16)] = zero16
        return carry

    lax.fori_loop(0, 256, _zero, 0)
    for cp in pre:
        cp.wait()

    inv = 1.0 / DELTA
    c0 = R / DELTA + 0.5
    scale = -500.0 * DELTA

    def _accum(h, sw, bv):
        u = h * inv + c0
        jf = jnp.minimum(jnp.maximum(u, 0.0), 15.0)
        j = jf.astype(jnp.int32)
        jq = j.astype(jnp.float32)
        z = jq * scale + (h * 500.0 + 500.0 * R)
        z = jnp.minimum(z, 30.0)
        wsig = sw / (1.0 + jnp.exp(z))
        idx = bv.astype(jnp.int32) + j * 16 + lane
        plsc.addupdate_scatter(hist_v, [idx], wsig)
        plsc.addupdate_scatter(hist_v, [idx + 2048], sw)

    def _fire(ch, r0, r1, sem):
        cps = []
        for a in range(5):
            cps.append(pltpu.async_copy(
                packed_hbm.at[i0_v.at[ch, a]],
                r0.at[pl.ds(a * E_SUB, E_SUB)], sem))
            cps.append(pltpu.async_copy(
                packed_hbm.at[i1_v.at[ch, a]],
                r1.at[pl.ds(a * E_SUB, E_SUB)], sem))
        return cps

    def _edge_compute(r0, r1):
        @plsc.parallel_loop(0, E_CHUNK, 1, unroll=8)
        def _ebody(e):
            h = jnp.minimum(r0[e, 0:16], r1[e, 0:16])
            s0 = r0[e, 16:32]
            wm = jnp.maximum(s0, r1[e, 16:32])
            zi = jnp.zeros((16,), jnp.int32)
            wv = wm.at[zi].get(mode="promise_in_bounds")
            bv = s0.at[zi + 1].get(mode="promise_in_bounds")
            _accum(h, -wv, bv)

    def _drain(r0, r1, sem):
        pltpu.make_async_copy(packed_hbm.at[pl.ds(0, E_CHUNK)], r0,
                              sem).wait()
        pltpu.make_async_copy(packed_hbm.at[pl.ds(0, E_CHUNK)], r1,
                              sem).wait()

    _fire(0, r0a_v, r1a_v, sem_a)
    _fire(1, r0b_v, r1b_v, sem_b)
    npairs = E_CHUNKS // 2

    def _pair(g, carry):
        _drain(r0a_v, r1a_v, sem_a)
        _edge_compute(r0a_v, r1a_v)

        @pl.when(g < npairs - 1)
        def _():
            _fire(2 * g + 2, r0a_v, r1a_v, sem_a)

        _drain(r0b_v, r1b_v, sem_b)
        _edge_compute(r0b_v, r1b_v)

        @pl.when(g < npairs - 1)
        def _():
            _fire(2 * g + 3, r0b_v, r1b_v, sem_b)

        return carry

    lax.fori_loop(0, npairs, _pair, 0)

    cps = []
    for a in range(5):
        for fv, rv in ((f0_v, r0a_v), (f1_v, r1a_v), (f2_v, r0b_v)):
            cps.append(pltpu.async_copy(
                packed_hbm.at[fv.at[a]],
                rv.at[pl.ds(a * E_SUB, E_SUB)], sem_a))
    for cp in cps:
        cp.wait()

    @plsc.parallel_loop(0, F_CHUNK, 1, unroll=8)
    def _fbody(e):
        h = jnp.minimum(jnp.minimum(r0a_v[e, 0:16], r1a_v[e, 0:16]),
                        r0b_v[e, 0:16])
        s0 = r0a_v[e, 16:32]
        wm = jnp.maximum(jnp.maximum(s0, r1a_v[e, 16:32]), r0b_v[e, 16:32])
        zi = jnp.zeros((16,), jnp.int32)
        wv = wm.at[zi].get(mode="promise_in_bounds")
        bv = s0.at[zi + 1].get(mode="promise_in_bounds")
        _accum(h, wv, bv)

    @plsc.parallel_loop(0, N_CHUNK, 1, unroll=8)
    def _nbody(e):
        h = node_v[e, 0:16]
        s0 = node_v[e, 16:32]
        zi = jnp.zeros((16,), jnp.int32)
        wv = s0.at[zi].get(mode="promise_in_bounds")
        bv = s0.at[zi + 1].get(mode="promise_in_bounds")
        _accum(h, wv, bv)

    def _stage(i, carry):
        stage_v[i, :] = hist_v[pl.ds(i * 16, 16)]
        return carry

    lax.fori_loop(0, 256, _stage, 0)
    pltpu.sync_copy(stage_v, out_hbm.at[wid])


_sc_call = pl.kernel(
    _sc_body,
    out_type=jax.ShapeDtypeStruct((NW, 256, 16), jnp.float32),
    mesh=plsc.VectorSubcoreMesh(core_axis_name="c", subcore_axis_name="s"),
    compiler_params=pltpu.CompilerParams(needs_layout_passes=False,
                                         use_tc_tiling_on_sc=False),
    scratch_types=[
        pltpu.VMEM((4096,), jnp.float32),
        pltpu.VMEM((256, 16), jnp.float32),
        pltpu.VMEM((E_CHUNK, 32), jnp.float32),
        pltpu.VMEM((E_CHUNK, 32), jnp.float32),
        pltpu.VMEM((E_CHUNK, 32), jnp.float32),
        pltpu.VMEM((E_CHUNK, 32), jnp.float32),
        pltpu.VMEM((N_CHUNK, 32), jnp.float32),
        pltpu.VMEM((E_CHUNKS, 5, E_SUB), jnp.int32),
        pltpu.VMEM((E_CHUNKS, 5, E_SUB), jnp.int32),
        pltpu.VMEM((5, E_SUB), jnp.int32),
        pltpu.VMEM((5, E_SUB), jnp.int32),
        pltpu.VMEM((5, E_SUB), jnp.int32),
        pltpu.SemaphoreType.DMA,
        pltpu.SemaphoreType.DMA,
        pltpu.SemaphoreType.DMA,
    ],
)


@jax.jit
def kernel(x, edge_index, face, node_weights, batch, v):
    packed = pl.pallas_call(
        _pack_body,
        out_shape=jax.ShapeDtypeStruct((N_PAD, 32), jnp.float32),
    )(x, v, node_weights[:, None], batch.astype(jnp.int32)[:, None])

    ei = edge_index.astype(jnp.int32).reshape(2 * NW * E_CHUNKS, 5, E_SUB)
    fa = face.astype(jnp.int32).reshape(3 * NW, 5, E_SUB)

    hist = _sc_call(packed, ei, fa)

    out2 = pl.pallas_call(
        _fin_body,
        out_shape=jax.ShapeDtypeStruct((128, 16), jnp.float32),
    )(hist)
    return out2.reshape(B, S, T)


# speedup vs baseline: 5.4632x; 1.0090x over previous
import functools

import jax
import jax.numpy as jnp
from jax import lax
from jax.experimental import pallas as pl
from jax.experimental.pallas import tpu as pltpu
from jax.experimental.pallas import tpu_sc as plsc

N = 10000
E = 160000
F = 20000
D = 3
T = 16
S = 16
R = 1.1
B = 8

DELTA = 2.0 * R / (S - 1)
NW = 32
N_PAD = 10240
E_CHUNK = 625
E_CHUNKS = 8
F_CHUNK = 625
N_CHUNK = 320
E_SUB = 125


def _pack_body(x_ref, v_ref, w_ref, b_ref, out_ref):
    nh = jnp.dot(x_ref[...], v_ref[...],
                 preferred_element_type=jnp.float32)
    row = jnp.concatenate(
        [nh, w_ref[...], b_ref[...].astype(jnp.float32) * 256.0,
         jnp.zeros((N, 14), jnp.float32)], axis=1)
    out_ref[0:N, :] = row
    out_ref[N:N_PAD, :] = jnp.zeros((N_PAD - N, 32), jnp.float32)


def _fin_body(hist_ref, out_ref):
    s = jnp.sum(hist_ref[...], axis=0)
    h2 = s[0:128, :]
    g2 = s[128:256, :]
    r = lax.broadcasted_iota(jnp.int32, (128, 128), 0)
    c = lax.broadcasted_iota(jnp.int32, (128, 128), 1)
    m = ((r >> 4) == (c >> 4)) & ((c & 15) < (r & 15))
    out_ref[...] = h2 + jnp.dot(m.astype(jnp.float32), g2,
                                preferred_element_type=jnp.float32)


def _sc_body(packed_hbm, ei_hbm, fa_hbm,
             out_hbm, hist_v, stage_v, r0a_v, r1a_v, r0b_v, r1b_v,
             node_v, i0_v, i1_v, f0_v, f1_v, f2_v, sem_i, sem_a, sem_b):
    cid = lax.axis_index("c")
    sid = lax.axis_index("s")
    wid = sid * 2 + cid
    lane = lax.iota(jnp.int32, 16)

    eibase = pl.multiple_of(wid * E_CHUNKS, 8)
    pre = [
        pltpu.async_copy(ei_hbm.at[pl.ds(eibase, E_CHUNKS)], i0_v, sem_i),
        pltpu.async_copy(ei_hbm.at[pl.ds(eibase + NW * E_CHUNKS, E_CHUNKS)],
                         i1_v, sem_i),
        pltpu.async_copy(fa_hbm.at[wid], f0_v, sem_i),
        pltpu.async_copy(fa_hbm.at[wid + NW], f1_v, sem_i),
        pltpu.async_copy(fa_hbm.at[wid + 2 * NW], f2_v, sem_i),
        pltpu.async_copy(
            packed_hbm.at[pl.ds(pl.multiple_of(wid * N_CHUNK, N_CHUNK),
                                N_CHUNK)], node_v, sem_i),
    ]

    zero16 = jnp.zeros((16,), jnp.float32)

    @plsc.parallel_loop(0, 256, 1, unroll=8)
    def _zero(i):
        hist_v[pl.ds(i * 16, 16)] = zero16

    for cp in pre:
        cp.wait()

    inv = 1.0 / DELTA
    c0 = R / DELTA + 0.5
    scale = -500.0 * DELTA

    def _accum(h, sw, bv):
        u = h * inv + c0
        jf = jnp.minimum(jnp.maximum(u, 0.0), 15.0)
        j = jf.astype(jnp.int32)
        jq = j.astype(jnp.float32)
        z = jq * scale + (h * 500.0 + 500.0 * R)
        z = jnp.minimum(z, 30.0)
        wsig = sw / (1.0 + jnp.exp(z))
        idx = bv.astype(jnp.int32) + j * 16 + lane
        plsc.addupdate_scatter(hist_v, [idx], wsig)
        plsc.addupdate_scatter(hist_v, [idx + 2048], sw)

    def _fire(ch, r0, r1, sem):
        cps = []
        for a in range(5):
            cps.append(pltpu.async_copy(
                packed_hbm.at[i0_v.at[ch, a]],
                r0.at[pl.ds(a * E_SUB, E_SUB)], sem))
            cps.append(pltpu.async_copy(
                packed_hbm.at[i1_v.at[ch, a]],
                r1.at[pl.ds(a * E_SUB, E_SUB)], sem))
        return cps

    def _edge_compute(r0, r1):
        @plsc.parallel_loop(0, E_CHUNK, 1, unroll=8)
        def _ebody(e):
            h = jnp.minimum(r0[e, 0:16], r1[e, 0:16])
            s0 = r0[e, 16:32]
            wm = jnp.maximum(s0, r1[e, 16:32])
            zi = jnp.zeros((16,), jnp.int32)
            wv = wm.at[zi].get(mode="promise_in_bounds")
            bv = s0.at[zi + 1].get(mode="promise_in_bounds")
            _accum(h, -wv, bv)

    def _drain(r0, r1, sem):
        pltpu.make_async_copy(packed_hbm.at[pl.ds(0, E_CHUNK)], r0,
                              sem).wait()
        pltpu.make_async_copy(packed_hbm.at[pl.ds(0, E_CHUNK)], r1,
                              sem).wait()

    _fire(0, r0a_v, r1a_v, sem_a)
    _fire(1, r0b_v, r1b_v, sem_b)
    npairs = E_CHUNKS // 2

    def _pair(g, carry):
        _drain(r0a_v, r1a_v, sem_a)
        _edge_compute(r0a_v, r1a_v)

        @pl.when(g < npairs - 1)
        def _():
            _fire(2 * g + 2, r0a_v, r1a_v, sem_a)

        _drain(r0b_v, r1b_v, sem_b)
        _edge_compute(r0b_v, r1b_v)

        @pl.when(g < npairs - 1)
        def _():
            _fire(2 * g + 3, r0b_v, r1b_v, sem_b)

        return carry

    lax.fori_loop(0, npairs, _pair, 0)

    cps = []
    for a in range(5):
        for fv, rv in ((f0_v, r0a_v), (f1_v, r1a_v), (f2_v, r0b_v)):
            cps.append(pltpu.async_copy(
                packed_hbm.at[fv.at[a]],
                rv.at[pl.ds(a * E_SUB, E_SUB)], sem_a))
    for cp in cps:
        cp.wait()

    @plsc.parallel_loop(0, F_CHUNK, 1, unroll=8)
    def _fbody(e):
        h = jnp.minimum(jnp.minimum(r0a_v[e, 0:16], r1a_v[e, 0:16]),
                        r0b_v[e, 0:16])
        s0 = r0a_v[e, 16:32]
        wm = jnp.maximum(jnp.maximum(s0, r1a_v[e, 16:32]), r0b_v[e, 16:32])
        zi = jnp.zeros((16,), jnp.int32)
        wv = wm.at[zi].get(mode="promise_in_bounds")
        bv = s0.at[zi + 1].get(mode="promise_in_bounds")
        _accum(h, wv, bv)

    @plsc.parallel_loop(0, N_CHUNK, 1, unroll=8)
    def _nbody(e):
        h = node_v[e, 0:16]
        s0 = node_v[e, 16:32]
        zi = jnp.zeros((16,), jnp.int32)
        wv = s0.at[zi].get(mode="promise_in_bounds")
        bv = s0.at[zi + 1].get(mode="promise_in_bounds")
        _accum(h, wv, bv)

    @plsc.parallel_loop(0, 256, 1, unroll=8)
    def _stage(i):
        stage_v[i, :] = hist_v[pl.ds(i * 16, 16)]

    pltpu.sync_copy(stage_v, out_hbm.at[wid])


_sc_call = pl.kernel(
    _sc_body,
    out_type=jax.ShapeDtypeStruct((NW, 256, 16), jnp.float32),
    mesh=plsc.VectorSubcoreMesh(core_axis_name="c", subcore_axis_name="s"),
    compiler_params=pltpu.CompilerParams(needs_layout_passes=False,
                                         use_tc_tiling_on_sc=False),
    scratch_types=[
        pltpu.VMEM((4096,), jnp.float32),
        pltpu.VMEM((256, 16), jnp.float32),
        pltpu.VMEM((E_CHUNK, 32), jnp.float32),
        pltpu.VMEM((E_CHUNK, 32), jnp.float32),
        pltpu.VMEM((E_CHUNK, 32), jnp.float32),
        pltpu.VMEM((E_CHUNK, 32), jnp.float32),
        pltpu.VMEM((N_CHUNK, 32), jnp.float32),
        pltpu.VMEM((E_CHUNKS, 5, E_SUB), jnp.int32),
        pltpu.VMEM((E_CHUNKS, 5, E_SUB), jnp.int32),
        pltpu.VMEM((5, E_SUB), jnp.int32),
        pltpu.VMEM((5, E_SUB), jnp.int32),
        pltpu.VMEM((5, E_SUB), jnp.int32),
        pltpu.SemaphoreType.DMA,
        pltpu.SemaphoreType.DMA,
        pltpu.SemaphoreType.DMA,
    ],
)


@jax.jit
def kernel(x, edge_index, face, node_weights, batch, v):
    packed = pl.pallas_call(
        _pack_body,
        out_shape=jax.ShapeDtypeStruct((N_PAD, 32), jnp.float32),
    )(x, v, node_weights[:, None], batch.astype(jnp.int32)[:, None])

    ei = edge_index.astype(jnp.int32).reshape(2 * NW * E_CHUNKS, 5, E_SUB)
    fa = face.astype(jnp.int32).reshape(3 * NW, 5, E_SUB)

    hist = _sc_call(packed, ei, fa)

    out2 = pl.pallas_call(
        _fin_body,
        out_shape=jax.ShapeDtypeStruct((128, 16), jnp.float32),
    )(hist)
    return out2.reshape(B, S, T)


# speedup vs baseline: 5.4815x; 1.0033x over previous
import functools

import jax
import jax.numpy as jnp
from jax import lax
from jax.experimental import pallas as pl
from jax.experimental.pallas import tpu as pltpu
from jax.experimental.pallas import tpu_sc as plsc

N = 10000
E = 160000
F = 20000
D = 3
T = 16
S = 16
R = 1.1
B = 8

DELTA = 2.0 * R / (S - 1)
NW = 32
N_PAD = 10240
E_CHUNK = 625
E_CHUNKS = 8
F_CHUNK = 625
N_CHUNK = 320
E_SUB = 125


def _pack_body(x_ref, v_ref, w_ref, b_ref, out_ref):
    nh = jnp.dot(x_ref[...], v_ref[...],
                 preferred_element_type=jnp.float32)
    row = jnp.concatenate(
        [nh, w_ref[...], b_ref[...].astype(jnp.float32) * 256.0,
         jnp.zeros((N, 14), jnp.float32)], axis=1)
    out_ref[0:N, :] = row
    out_ref[N:N_PAD, :] = jnp.zeros((N_PAD - N, 32), jnp.float32)


def _fin_body(hist_ref, out_ref):
    s = jnp.sum(hist_ref[...], axis=0)
    h2 = s[0:128, :]
    g2 = s[128:256, :]
    r = lax.broadcasted_iota(jnp.int32, (128, 128), 0)
    c = lax.broadcasted_iota(jnp.int32, (128, 128), 1)
    m = ((r >> 4) == (c >> 4)) & ((c & 15) < (r & 15))
    out_ref[...] = h2 + jnp.dot(m.astype(jnp.float32), g2,
                                preferred_element_type=jnp.float32)


def _sc_body(packed_hbm, ei_hbm, fa_hbm,
             out_hbm, hist_v, stage_v, r0a_v, r1a_v, r0b_v, r1b_v,
             node_v, i0_v, i1_v, f0_v, f1_v, f2_v, sem_i, sem_a, sem_b):
    cid = lax.axis_index("c")
    sid = lax.axis_index("s")
    wid = sid * 2 + cid
    lane = lax.iota(jnp.int32, 16)

    eibase = pl.multiple_of(wid * E_CHUNKS, 8)
    pre = [
        pltpu.async_copy(ei_hbm.at[pl.ds(eibase, E_CHUNKS)], i0_v, sem_i),
        pltpu.async_copy(ei_hbm.at[pl.ds(eibase + NW * E_CHUNKS, E_CHUNKS)],
                         i1_v, sem_i),
        pltpu.async_copy(fa_hbm.at[wid], f0_v, sem_i),
        pltpu.async_copy(fa_hbm.at[wid + NW], f1_v, sem_i),
        pltpu.async_copy(fa_hbm.at[wid + 2 * NW], f2_v, sem_i),
        pltpu.async_copy(
            packed_hbm.at[pl.ds(pl.multiple_of(wid * N_CHUNK, N_CHUNK),
                                N_CHUNK)], node_v, sem_i),
    ]

    zero16 = jnp.zeros((16,), jnp.float32)

    @plsc.parallel_loop(0, 256, 1, unroll=8)
    def _zero(i):
        hist_v[pl.ds(i * 16, 16)] = zero16

    for cp in pre:
        cp.wait()

    inv = 1.0 / DELTA
    c0 = R / DELTA + 0.5
    scale = -500.0 * DELTA

    def _accum(h, sw, bv):
        u = h * inv + c0
        jf = jnp.minimum(jnp.maximum(u, 0.0), 15.0)
        j = jf.astype(jnp.int32)
        jq = j.astype(jnp.float32)
        z = jq * scale + (h * 500.0 + 500.0 * R)
        z = jnp.minimum(z, 30.0)
        wsig = sw / (1.0 + jnp.exp(z))
        idx = bv.astype(jnp.int32) + j * 16 + lane
        plsc.addupdate_scatter(hist_v, [idx], wsig)
        plsc.addupdate_scatter(hist_v, [idx + 2048], sw)

    def _fire(ch, r0, r1, sem):
        cps = []
        for a in range(5):
            cps.append(pltpu.async_copy(
                packed_hbm.at[i0_v.at[ch, a]],
                r0.at[pl.ds(a * E_SUB, E_SUB)], sem))
            cps.append(pltpu.async_copy(
                packed_hbm.at[i1_v.at[ch, a]],
                r1.at[pl.ds(a * E_SUB, E_SUB)], sem))
        return cps

    def _edge_compute(r0, r1):
        @plsc.parallel_loop(0, E_CHUNK, 1, unroll=8)
        def _ebody(e):
            h = jnp.minimum(r0[e, 0:16], r1[e, 0:16])
            s0 = r0[e, 16:32]
            wm = jnp.maximum(s0, r1[e, 16:32])
            wv = jnp.full((16,), wm[0], jnp.float32)
            bv = jnp.full((16,), s0[1], jnp.float32)
            _accum(h, -wv, bv)

    def _drain(r0, r1, sem):
        pltpu.make_async_copy(packed_hbm.at[pl.ds(0, E_CHUNK)], r0,
                              sem).wait()
        pltpu.make_async_copy(packed_hbm.at[pl.ds(0, E_CHUNK)], r1,
                              sem).wait()

    _fire(0, r0a_v, r1a_v, sem_a)
    _fire(1, r0b_v, r1b_v, sem_b)
    npairs = E_CHUNKS // 2

    def _pair(g, carry):
        _drain(r0a_v, r1a_v, sem_a)
        _edge_compute(r0a_v, r1a_v)

        @pl.when(g < npairs - 1)
        def _():
            _fire(2 * g + 2, r0a_v, r1a_v, sem_a)

        _drain(r0b_v, r1b_v, sem_b)
        _edge_compute(r0b_v, r1b_v)

        @pl.when(g < npairs - 1)
        def _():
            _fire(2 * g + 3, r0b_v, r1b_v, sem_b)

        return carry

    lax.fori_loop(0, npairs, _pair, 0)

    cps = []
    for a in range(5):
        for fv, rv in ((f0_v, r0a_v), (f1_v, r1a_v), (f2_v, r0b_v)):
            cps.append(pltpu.async_copy(
                packed_hbm.at[fv.at[a]],
                rv.at[pl.ds(a * E_SUB, E_SUB)], sem_a))
    for cp in cps:
        cp.wait()

    @plsc.parallel_loop(0, F_CHUNK, 1, unroll=8)
    def _fbody(e):
        h = jnp.minimum(jnp.minimum(r0a_v[e, 0:16], r1a_v[e, 0:16]),
                        r0b_v[e, 0:16])
        s0 = r0a_v[e, 16:32]
        wm = jnp.maximum(jnp.maximum(s0, r1a_v[e, 16:32]), r0b_v[e, 16:32])
        zi = jnp.zeros((16,), jnp.int32)
        wv = wm.at[zi].get(mode="promise_in_bounds")
        bv = s0.at[zi + 1].get(mode="promise_in_bounds")
        _accum(h, wv, bv)

    @plsc.parallel_loop(0, N_CHUNK, 1, unroll=8)
    def _nbody(e):
        h = node_v[e, 0:16]
        s0 = node_v[e, 16:32]
        zi = jnp.zeros((16,), jnp.int32)
        wv = s0.at[zi].get(mode="promise_in_bounds")
        bv = s0.at[zi + 1].get(mode="promise_in_bounds")
        _accum(h, wv, bv)

    @plsc.parallel_loop(0, 256, 1, unroll=8)
    def _stage(i):
        stage_v[i, :] = hist_v[pl.ds(i * 16, 16)]

    pltpu.sync_copy(stage_v, out_hbm.at[wid])


_sc_call = pl.kernel(
    _sc_body,
    out_type=jax.ShapeDtypeStruct((NW, 256, 16), jnp.float32),
    mesh=plsc.VectorSubcoreMesh(core_axis_name="c", subcore_axis_name="s"),
    compiler_params=pltpu.CompilerParams(needs_layout_passes=False,
                                         use_tc_tiling_on_sc=False),
    scratch_types=[
        pltpu.VMEM((4096,), jnp.float32),
        pltpu.VMEM((256, 16), jnp.float32),
        pltpu.VMEM((E_CHUNK, 32), jnp.float32),
        pltpu.VMEM((E_CHUNK, 32), jnp.float32),
        pltpu.VMEM((E_CHUNK, 32), jnp.float32),
        pltpu.VMEM((E_CHUNK, 32), jnp.float32),
        pltpu.VMEM((N_CHUNK, 32), jnp.float32),
        pltpu.VMEM((E_CHUNKS, 5, E_SUB), jnp.int32),
        pltpu.VMEM((E_CHUNKS, 5, E_SUB), jnp.int32),
        pltpu.VMEM((5, E_SUB), jnp.int32),
        pltpu.VMEM((5, E_SUB), jnp.int32),
        pltpu.VMEM((5, E_SUB), jnp.int32),
        pltpu.SemaphoreType.DMA,
        pltpu.SemaphoreType.DMA,
        pltpu.SemaphoreType.DMA,
    ],
)


@jax.jit
def kernel(x, edge_index, face, node_weights, batch, v):
    packed = pl.pallas_call(
        _pack_body,
        out_shape=jax.ShapeDtypeStruct((N_PAD, 32), jnp.float32),
    )(x, v, node_weights[:, None], batch.astype(jnp.int32)[:, None])

    ei = edge_index.astype(jnp.int32).reshape(2 * NW * E_CHUNKS, 5, E_SUB)
    fa = face.astype(jnp.int32).reshape(3 * NW, 5, E_SUB)

    hist = _sc_call(packed, ei, fa)

    out2 = pl.pallas_call(
        _fin_body,
        out_shape=jax.ShapeDtypeStruct((128, 16), jnp.float32),
    )(hist)
    return out2.reshape(B, S, T)
